# baseline (speedup 1.0000x reference)
import jax
import jax.numpy as jnp
from jax import lax
from jax.experimental import pallas as pl
from jax.experimental.pallas import tpu as pltpu
from jax.experimental.pallas import tpu_sc as plsc

_NC = 2
_NS = 16
_L = 16
_NW = _NC * _NS

_B, _S, _D = 4096, 200, 64
_F = _S * _D
_RPW = _B // _NW
_C = 4


def _body(x_hbm, pos_hbm, out_hbm, pos_v, buf, sem):
    wid = lax.axis_index("s") * _NC + lax.axis_index("c")
    base = wid * _RPW
    pltpu.sync_copy(pos_hbm, pos_v)

    def chunk(g, carry):
        row0 = base + g * _C
        pltpu.sync_copy(x_hbm.at[pl.ds(row0, _C)], buf)

        def add_i(i, c2):
            off = i * _L
            p = pos_v[pl.ds(off, _L)]
            for c in range(_C):
                buf[c, pl.ds(off, _L)] = buf[c, pl.ds(off, _L)] + p
            return c2

        lax.fori_loop(0, _F // _L, add_i, 0, unroll=4)
        pltpu.sync_copy(buf, out_hbm.at[pl.ds(row0, _C)])
        return carry

    lax.fori_loop(0, _RPW // _C, chunk, 0)


def _tc_body(x_ref, pos_ref, o_ref):
    o_ref[...] = x_ref[...] + pos_ref[...]


_TCBLK = 32


def _tc_add(x, pos):
    return pl.pallas_call(
        _tc_body,
        grid=(_B // _TCBLK,),
        in_specs=[
            pl.BlockSpec((_TCBLK, _S, _D), lambda i: (i, 0, 0)),
            pl.BlockSpec((1, _S, _D), lambda i: (0, 0, 0)),
        ],
        out_specs=pl.BlockSpec((_TCBLK, _S, _D), lambda i: (i, 0, 0)),
        out_shape=jax.ShapeDtypeStruct((_B, _S, _D), jnp.float32),
    )(x, pos)


def kernel(x, pos_embedding):
    return _tc_add(x, pos_embedding.reshape(1, _S, _D))


def _sc_kernel_unused(x, pos_embedding):
    xf = x.reshape(_B, _F)
    posf = pos_embedding.reshape(_F)
    mesh = plsc.VectorSubcoreMesh(core_axis_name="c", subcore_axis_name="s")
    out = pl.kernel(
        _body,
        out_type=jax.ShapeDtypeStruct((_B, _F), jnp.float32),
        mesh=mesh,
        scratch_types=[
            pltpu.VMEM((_F,), jnp.float32),
            pltpu.VMEM((_C, _F), jnp.float32),
            pltpu.SemaphoreType.DMA,
        ],
    )(xf, posf)
    return out.reshape(_B, _S, _D)


# speedup vs baseline: 1.5579x; 1.5579x over previous
import jax
import jax.numpy as jnp
from jax import lax
from jax.experimental import pallas as pl
from jax.experimental.pallas import tpu as pltpu
from jax.experimental.pallas import tpu_sc as plsc

_NC = 2
_NS = 16
_L = 16
_NW = _NC * _NS

_B, _S, _D = 4096, 200, 64
_F = _S * _D
_RPW = _B // _NW
_C = 2
_G = _RPW // _C


def _body(x_hbm, pos_hbm, out_hbm, pos_v, in0, in1, ou0, ou1,
          si0, si1, so0, so1):
    wid = lax.axis_index("s") * _NC + lax.axis_index("c")
    base = wid * _RPW
    pltpu.sync_copy(pos_hbm, pos_v)

    ins, outs = (in0, in1), (ou0, ou1)
    sis, sos = (si0, si1), (so0, so1)

    pltpu.async_copy(x_hbm.at[pl.ds(base, _C)], in0, si0)
    pltpu.async_copy(x_hbm.at[pl.ds(base + _C, _C)], in1, si1)

    def step(g2, carry):
        for b in range(2):
            g = g2 * 2 + b
            row0 = base + g * _C
            ib, ob, si, so = ins[b], outs[b], sis[b], sos[b]

            pltpu.make_async_copy(x_hbm.at[pl.ds(row0, _C)], ib, si).wait()

            @pl.when(g2 >= 1)
            def _wait_out():
                pltpu.make_async_copy(
                    ob, out_hbm.at[pl.ds(row0, _C)], so).wait()

            @plsc.parallel_loop(0, _F, step=_L, unroll=8)
            def _add(off):
                p = pos_v[pl.ds(off, _L)]
                for c in range(_C):
                    ob[c, pl.ds(off, _L)] = ib[c, pl.ds(off, _L)] + p

            pltpu.async_copy(ob, out_hbm.at[pl.ds(row0, _C)], so)

            @pl.when(g2 < _G // 2 - 1)
            def _start_next_in():
                pltpu.async_copy(
                    x_hbm.at[pl.ds(row0 + 2 * _C, _C)], ib, si)

        return carry

    lax.fori_loop(0, _G // 2, step, 0)

    pltpu.make_async_copy(ou0, out_hbm.at[pl.ds(base, _C)], so0).wait()
    pltpu.make_async_copy(ou1, out_hbm.at[pl.ds(base, _C)], so1).wait()


def kernel(x, pos_embedding):
    xf = x.reshape(_B, _F)
    posf = pos_embedding.reshape(_F)
    mesh = plsc.VectorSubcoreMesh(core_axis_name="c", subcore_axis_name="s")
    out = pl.kernel(
        _body,
        out_type=jax.ShapeDtypeStruct((_B, _F), jnp.float32),
        mesh=mesh,
        scratch_types=[
            pltpu.VMEM((_F,), jnp.float32),
            pltpu.VMEM((_C, _F), jnp.float32),
            pltpu.VMEM((_C, _F), jnp.float32),
            pltpu.VMEM((_C, _F), jnp.float32),
            pltpu.VMEM((_C, _F), jnp.float32),
            pltpu.SemaphoreType.DMA,
            pltpu.SemaphoreType.DMA,
            pltpu.SemaphoreType.DMA,
            pltpu.SemaphoreType.DMA,
        ],
    )(xf, posf)
    return out.reshape(_B, _S, _D)


# speedup vs baseline: 4.7026x; 3.0185x over previous
import jax
import jax.numpy as jnp
from jax import lax
from jax.experimental import pallas as pl
from jax.experimental.pallas import tpu as pltpu
from jax.experimental.pallas import tpu_sc as plsc

_NC = 2
_NS = 16
_L = 16
_NW = _NC * _NS

_B, _S, _D = 4096, 200, 64
_R = _S * _D
_RPW = _R // _NW
_CR = 8
_NCOL = 2
_CB = _B // _NCOL
_G = (_RPW // _CR) * _NCOL
_NBUF = 6
_GMAIN = (_G // _NBUF) * _NBUF
_PREF = 4


def _body(x_hbm, pos_hbm, out_hbm, pos_v, b0, b1, b2, b3, b4, b5,
          si0, si1, si2, si3, si4, si5, so0, so1, so2, so3, so4, so5):
    wid = lax.axis_index("s") * _NC + lax.axis_index("c")
    base = wid * _RPW
    pltpu.sync_copy(pos_hbm, pos_v.at[pl.ds(0, _R)])

    bufs = (b0, b1, b2, b3, b4, b5)
    sis = (si0, si1, si2, si3, si4, si5)
    sos = (so0, so1, so2, so3, so4, so5)

    def chunk_slice(ref, c):
        row0 = base + (c // _NCOL) * _CR
        col0 = (c % _NCOL) * _CB
        return ref.at[pl.ds(row0, _CR), pl.ds(col0, _CB)]

    for k in range(_PREF):
        pltpu.async_copy(chunk_slice(x_hbm, k), bufs[k], sis[k])

    def process(c, b, in_main_loop):
        buf, si, so = bufs[b], sis[b], sos[b]

        pltpu.make_async_copy(chunk_slice(x_hbm, c), buf, si).wait()

        pv = pos_v[pl.ds(base + (c // _NCOL) * _CR, _L)]
        for r in range(_CR):
            p = jnp.broadcast_to(pv[r], (_L,))

            @plsc.parallel_loop(0, _CB, step=_L, unroll=16)
            def _add(off):
                buf[r, pl.ds(off, _L)] = buf[r, pl.ds(off, _L)] + p

        pltpu.async_copy(buf, chunk_slice(out_hbm, c), so)

        if in_main_loop:
            nb = (b + _PREF) % _NBUF

            @pl.when(c + _PREF < _G)
            def _start_next_in():
                @pl.when(c >= _NBUF - _PREF)
                def _wait_prev_out():
                    pltpu.make_async_copy(
                        bufs[nb], chunk_slice(out_hbm, c), sos[nb]).wait()

                pltpu.async_copy(
                    chunk_slice(x_hbm, c + _PREF), bufs[nb], sis[nb])

    def step(c6, carry):
        for b in range(_NBUF):
            process(c6 * _NBUF + b, b, True)
        return carry

    lax.fori_loop(0, _GMAIN // _NBUF, step, 0)

    for c in range(_GMAIN, _G):
        process(c, c % _NBUF, False)

    for k in range(_G - _NBUF, _G):
        pltpu.make_async_copy(
            bufs[k % _NBUF], chunk_slice(out_hbm, k), sos[k % _NBUF]).wait()


def kernel(x, pos_embedding):
    xp = x.transpose(1, 2, 0).reshape(_R, _B)
    posf = pos_embedding.reshape(_R)
    mesh = plsc.VectorSubcoreMesh(core_axis_name="c", subcore_axis_name="s")
    out = pl.kernel(
        _body,
        out_type=jax.ShapeDtypeStruct((_R, _B), jnp.float32),
        mesh=mesh,
        scratch_types=(
            [pltpu.VMEM((_R + _L,), jnp.float32)]
            + [pltpu.VMEM((_CR, _CB), jnp.float32) for _ in range(_NBUF)]
            + [pltpu.SemaphoreType.DMA for _ in range(2 * _NBUF)]
        ),
    )(xp, posf)
    return out.reshape(_S, _D, _B).transpose(2, 0, 1)


# speedup vs baseline: 4.7489x; 1.0098x over previous
import jax
import jax.numpy as jnp
from jax import lax
from jax.experimental import pallas as pl
from jax.experimental.pallas import tpu as pltpu
from jax.experimental.pallas import tpu_sc as plsc

_NC = 2
_NS = 16
_L = 16
_NW = _NC * _NS

_B, _S, _D = 4096, 200, 64
_R = _S * _D
_RPW = _R // _NW
_CR = 8
_NCOL = 1
_CB = _B // _NCOL
_G = (_RPW // _CR) * _NCOL
_NBUF = 3
_GMAIN = (_G // _NBUF) * _NBUF
_PREF = 2


def _body(x_hbm, pos_hbm, out_hbm, pos_v, *scr):
    wid = lax.axis_index("s") * _NC + lax.axis_index("c")
    base = wid * _RPW
    pltpu.sync_copy(pos_hbm, pos_v.at[pl.ds(0, _R)])

    bufs = scr[:_NBUF]
    sis = scr[_NBUF:2 * _NBUF]
    sos = scr[2 * _NBUF:3 * _NBUF]

    def chunk_slice(ref, c):
        row0 = base + (c // _NCOL) * _CR
        col0 = (c % _NCOL) * _CB
        return ref.at[pl.ds(row0, _CR), pl.ds(col0, _CB)]

    for k in range(_PREF):
        pltpu.async_copy(chunk_slice(x_hbm, k), bufs[k], sis[k])

    def process(c, b, in_main_loop):
        buf, si, so = bufs[b], sis[b], sos[b]

        pltpu.make_async_copy(chunk_slice(x_hbm, c), buf, si).wait()

        pv = pos_v[pl.ds(base + (c // _NCOL) * _CR, _L)]
        for r in range(_CR):
            p = jnp.broadcast_to(pv[r], (_L,))

            @plsc.parallel_loop(0, _CB, step=_L, unroll=16)
            def _add(off):
                buf[r, pl.ds(off, _L)] = buf[r, pl.ds(off, _L)] + p

        pltpu.async_copy(buf, chunk_slice(out_hbm, c), so)

        if in_main_loop:
            nb = (b + _PREF) % _NBUF

            @pl.when(c + _PREF < _G)
            def _start_next_in():
                @pl.when(c >= _NBUF - _PREF)
                def _wait_prev_out():
                    pltpu.make_async_copy(
                        bufs[nb], chunk_slice(out_hbm, c), sos[nb]).wait()

                pltpu.async_copy(
                    chunk_slice(x_hbm, c + _PREF), bufs[nb], sis[nb])

    def step(c6, carry):
        for b in range(_NBUF):
            process(c6 * _NBUF + b, b, True)
        return carry

    lax.fori_loop(0, _GMAIN // _NBUF, step, 0)

    for c in range(_GMAIN, _G):
        process(c, c % _NBUF, False)

    for k in range(_G - _NBUF, _G):
        pltpu.make_async_copy(
            bufs[k % _NBUF], chunk_slice(out_hbm, k), sos[k % _NBUF]).wait()


def kernel(x, pos_embedding):
    xp = x.transpose(1, 2, 0).reshape(_R, _B)
    posf = pos_embedding.reshape(_R)
    mesh = plsc.VectorSubcoreMesh(core_axis_name="c", subcore_axis_name="s")
    out = pl.kernel(
        _body,
        out_type=jax.ShapeDtypeStruct((_R, _B), jnp.float32),
        mesh=mesh,
        scratch_types=(
            [pltpu.VMEM((_R + _L,), jnp.float32)]
            + [pltpu.VMEM((_CR, _CB), jnp.float32) for _ in range(_NBUF)]
            + [pltpu.SemaphoreType.DMA for _ in range(2 * _NBUF)]
        ),
    )(xp, posf)
    return out.reshape(_S, _D, _B).transpose(2, 0, 1)
